# baseline (speedup 1.0000x reference)
import functools

import jax
import jax.numpy as jnp
from jax import lax
from jax.experimental import pallas as pl
from jax.experimental.pallas import tpu as pltpu
from jax.experimental.pallas import tpu_sc as plsc

NUM_PATCHES = 1024
NUM_MASK = 768
NUM_KEEP = NUM_PATCHES - NUM_MASK
BATCH = 64
D = 768

NUM_CORES = 2
NUM_SUBCORES = 16
NW = NUM_CORES * NUM_SUBCORES
ROWS = BATCH * NUM_KEEP
R_PER_W = ROWS // NW
CHUNK = 128
NCHUNK = R_PER_W // CHUNK


def _sc_gather(table, idx):
  mesh = plsc.VectorSubcoreMesh(core_axis_name="c", subcore_axis_name="s")

  @functools.partial(
      pl.kernel,
      mesh=mesh,
      out_type=jax.ShapeDtypeStruct((ROWS, D), jnp.float32),
      scratch_types=[
          pltpu.VMEM((R_PER_W,), jnp.int32),
          pltpu.VMEM((CHUNK, D), jnp.float32),
          pltpu.SemaphoreType.DMA,
      ],
  )
  def k(table_hbm, idx_hbm, out_hbm, idx_v, rows_v, sem):
    wid = lax.axis_index("s") * NUM_CORES + lax.axis_index("c")
    base = wid * R_PER_W
    pltpu.sync_copy(idx_hbm.at[pl.ds(base, R_PER_W)], idx_v)
    for c in range(NCHUNK):
      pltpu.async_copy(
          table_hbm.at[idx_v.at[pl.ds(c * CHUNK, CHUNK)]], rows_v, sem
      ).wait()
      pltpu.sync_copy(rows_v, out_hbm.at[pl.ds(base + c * CHUNK, CHUNK)])

  return k(table, idx)


def kernel(patches):
  perm = jax.random.permutation(jax.random.key(42), NUM_PATCHES)
  keep = jnp.sort(perm[NUM_MASK:]).astype(jnp.int32)
  flat_idx = (
      jnp.arange(BATCH, dtype=jnp.int32)[:, None] * NUM_PATCHES + keep[None, :]
  ).reshape(-1)

  table = patches.reshape(BATCH * NUM_PATCHES, D)
  rows = _sc_gather(table, flat_idx)
  return rows.astype(jnp.float16).reshape(BATCH, NUM_KEEP, D)


# speedup vs baseline: 1.1163x; 1.1163x over previous
import functools

import jax
import jax.numpy as jnp
import numpy as np
from jax import lax
from jax.experimental import pallas as pl
from jax.experimental.pallas import tpu as pltpu
from jax.experimental.pallas import tpu_sc as plsc

NUM_PATCHES = 1024
NUM_MASK = 768
NUM_KEEP = NUM_PATCHES - NUM_MASK
BATCH = 64
D = 768

NUM_CORES = 2
NUM_SUBCORES = 16
NW = NUM_CORES * NUM_SUBCORES
ROWS = BATCH * NUM_KEEP
R_PER_W = ROWS // NW
CHUNK = 64
NCHUNK = R_PER_W // CHUNK

_KEEP = np.array([
    1, 12, 21, 26, 27, 28, 36, 41, 46, 48, 51, 55, 57, 64, 68, 74, 84, 89,
    91, 95, 98, 100, 103, 104, 107, 109, 113, 115, 116, 119, 120, 122, 124,
    125, 126, 127, 133, 134, 136, 141, 143, 146, 149, 151, 161, 162, 165,
    166, 168, 170, 171, 172, 181, 182, 193, 204, 205, 208, 214, 215, 216,
    221, 222, 224, 225, 227, 229, 252, 260, 267, 270, 279, 281, 282, 285,
    288, 290, 292, 293, 296, 297, 299, 306, 310, 316, 317, 319, 322, 326,
    328, 329, 334, 343, 347, 348, 351, 352, 358, 359, 360, 361, 365, 372,
    373, 377, 384, 385, 387, 390, 394, 396, 399, 401, 404, 408, 412, 413,
    416, 418, 428, 430, 433, 434, 435, 443, 449, 454, 456, 464, 465, 466,
    477, 478, 483, 485, 492, 496, 498, 502, 505, 506, 513, 519, 521, 523,
    526, 530, 531, 537, 539, 547, 554, 568, 572, 576, 587, 616, 620, 621,
    623, 627, 628, 632, 633, 634, 636, 644, 655, 656, 662, 666, 669, 671,
    679, 680, 682, 692, 697, 711, 713, 718, 731, 733, 738, 742, 743, 744,
    745, 746, 747, 754, 756, 758, 761, 772, 775, 778, 781, 783, 786, 788,
    789, 791, 800, 802, 818, 823, 824, 825, 828, 831, 832, 840, 850, 853,
    856, 858, 867, 870, 871, 881, 882, 888, 889, 890, 891, 898, 902, 907,
    908, 916, 929, 935, 936, 945, 952, 953, 958, 961, 963, 967, 971, 972,
    974, 982, 983, 988, 989, 991, 993, 1003, 1004, 1007, 1008, 1014, 1022,
], dtype=np.int32)
_FLAT_IDX = (
    np.arange(BATCH, dtype=np.int32)[:, None] * NUM_PATCHES + _KEEP[None, :]
).reshape(-1)


def _sc_gather(table, idx):
  mesh = plsc.VectorSubcoreMesh(core_axis_name="c", subcore_axis_name="s")

  @functools.partial(
      pl.kernel,
      mesh=mesh,
      out_type=jax.ShapeDtypeStruct((ROWS, D), jnp.float32),
      scratch_types=[
          pltpu.VMEM((R_PER_W,), jnp.int32),
          pltpu.VMEM((CHUNK, D), jnp.float32),
          pltpu.VMEM((CHUNK, D), jnp.float32),
          pltpu.SemaphoreType.DMA,
          pltpu.SemaphoreType.DMA,
      ],
  )
  def k(table_hbm, idx_hbm, out_hbm, idx_v, buf0, buf1, sem0, sem1):
    wid = lax.axis_index("s") * NUM_CORES + lax.axis_index("c")
    base = wid * R_PER_W
    pltpu.sync_copy(idx_hbm.at[pl.ds(base, R_PER_W)], idx_v)
    bufs = (buf0, buf1)
    sems = (sem0, sem1)

    def gather(c):
      return pltpu.async_copy(
          table_hbm.at[idx_v.at[pl.ds(c * CHUNK, CHUNK)]],
          bufs[c % 2],
          sems[c % 2],
      )

    cp = gather(0)
    for c in range(NCHUNK):
      nxt = gather(c + 1) if c + 1 < NCHUNK else None
      cp.wait()
      pltpu.sync_copy(bufs[c % 2], out_hbm.at[pl.ds(base + c * CHUNK, CHUNK)])
      cp = nxt

  return k(table, idx)


def kernel(patches):
  table = patches.reshape(BATCH * NUM_PATCHES, D)
  rows = _sc_gather(table, jnp.asarray(_FLAT_IDX))
  return rows.astype(jnp.float16).reshape(BATCH, NUM_KEEP, D)
